# baseline (speedup 1.0000x reference)
import functools

import jax
import jax.numpy as jnp
from jax import lax
from jax.experimental import pallas as pl
from jax.experimental.pallas import tpu as pltpu
from jax.experimental.pallas import tpu_sc as plsc

_B, _L = 1024, 200
_N = _B * _L
_D = 50
_DP = 56
_GW = 128
_NC, _NS = 2, 16
_NW = _NC * _NS
_CHUNK = 128
_CPW = _N // (_NW * _CHUNK)
_TBLK = 512
_NB = _N // _TBLK
_PBLK = 131072
_RBLK = 256


def _flat_extend(t, outlen):
    nb = (outlen + _PBLK - 1) // _PBLK

    def body(i_ref, o_ref):
        o_ref[...] = i_ref[...]

    return pl.pallas_call(
        body,
        grid=(nb,),
        in_specs=[pl.BlockSpec((_PBLK,), lambda i: (i,))],
        out_specs=pl.BlockSpec((_PBLK,), lambda i: (i,)),
        out_shape=jax.ShapeDtypeStruct((outlen,), jnp.float32),
        compiler_params=pltpu.CompilerParams(
            dimension_semantics=("arbitrary",)),
    )(t)


def _repitch(t2):
    q = t2.shape[0]
    nb = (q + _RBLK - 1) // _RBLK

    def body(i_ref, o_ref):
        x = i_ref[...]
        z = jnp.zeros((_RBLK, _DP - _D), jnp.float32)
        pieces = []
        for a in range(8):
            pieces.append(x[:, 50 * a:50 * a + 50])
            pieces.append(z)
        o_ref[...] = jnp.concatenate(pieces, axis=1)

    return pl.pallas_call(
        body,
        grid=(nb,),
        in_specs=[pl.BlockSpec((_RBLK, 400), lambda i: (i, 0))],
        out_specs=pl.BlockSpec((_RBLK, 448), lambda i: (i, 0)),
        out_shape=jax.ShapeDtypeStruct((q, 448), jnp.float32),
        compiler_params=pltpu.CompilerParams(
            dimension_semantics=("arbitrary",)),
    )(t2)


def _prep_table(t):
    flat = t.reshape(-1)
    m = (flat.size + 399) // 400
    ext = _flat_extend(flat, m * 400).reshape(m, 400)
    return _repitch(ext).reshape(8 * m, _DP)


def _sc_gather(x0, x1, x4, t0, t1, t4):
    mesh = plsc.VectorSubcoreMesh(core_axis_name="c", subcore_axis_name="s")
    ot = [jax.ShapeDtypeStruct((_N, _GW), jnp.float32) for _ in range(3)]

    @functools.partial(
        pl.kernel, mesh=mesh, out_type=ot,
        compiler_params=pltpu.CompilerParams(use_tc_tiling_on_sc=False),
        scratch_types=[
            pltpu.VMEM((_CPW, _CHUNK), jnp.int32),
            pltpu.VMEM((2, _CHUNK, _DP), jnp.float32),
            pltpu.SemaphoreType.DMA,
        ],
    )
    def k(x0h, x1h, x4h, t0h, t1h, t4h, o0h, o1h, o4h,
          iv, bufs, sem):
        wid = lax.axis_index("s") * _NC + lax.axis_index("c")
        row0 = wid * _CPW
        for xh, th, ohbm in ((x0h, t0h, o0h), (x1h, t1h, o1h), (x4h, t4h, o4h)):
            pltpu.sync_copy(xh.at[wid], iv)

            def ga(j, b, th=th):
                return pltpu.make_async_copy(th.at[iv.at[j]], bufs.at[b], sem)

            ga(0, 0).start()

            def body(j, carry, ga=ga, ohbm=ohbm):
                b = lax.rem(j, 2)
                ga(j, b).wait()
                nxt = lax.min(j + 1, _CPW - 1)
                ga(nxt, 1 - b).start()
                pltpu.sync_copy(
                    bufs.at[b],
                    ohbm.at[pl.ds((row0 + j) * _CHUNK, _CHUNK),
                            pl.ds(0, _DP)])
                return carry

            lax.fori_loop(0, _CPW, body, 0)
            ga(_CPW - 1, _CPW % 2).wait()

    return k(x0, x1, x4, t0, t1, t4)


def _mlp(g0, g1, g4, x2r, x3r, a0, a1, a4, p23, b1, w2, b2, s2, bt2):
    def body(x2_ref, x3_ref, g0_ref, g1_ref, g4_ref, a0_ref, a1_ref, a4_ref,
             p23_ref, b1_ref, w2_ref, b2_ref, s2_ref, bt2_ref, o_ref):
        x2 = x2_ref[0, 0, :]
        x3 = x3_ref[0, 0, :]
        it = lax.broadcasted_iota(jnp.int32, (_TBLK, 16), 1)
        oh = jnp.logical_or(x2[:, None] == it,
                            (x3[:, None] + 5) == it).astype(jnp.float32)
        acc = jnp.dot(g0_ref[:, 0:_D], a0_ref[...],
                      preferred_element_type=jnp.float32)
        acc += jnp.dot(g1_ref[:, 0:_D], a1_ref[...],
                       preferred_element_type=jnp.float32)
        acc += jnp.dot(g4_ref[:, 0:_D], a4_ref[...],
                       preferred_element_type=jnp.float32)
        acc += jnp.dot(oh, p23_ref[...], preferred_element_type=jnp.float32)
        acc += b1_ref[...]
        h = jnp.where(acc > 0, acc, jnp.exp(acc) - 1.0)
        acc2 = jnp.dot(h, w2_ref[...], preferred_element_type=jnp.float32)
        acc2 += b2_ref[...]
        h2 = jnp.where(acc2 > 0, acc2, jnp.exp(acc2) - 1.0)
        o_ref[...] = h2 * s2_ref[...] + bt2_ref[...]

    full = lambda shape: pl.BlockSpec(shape, lambda i: tuple(0 for _ in shape))
    return pl.pallas_call(
        body,
        grid=(_NB,),
        in_specs=[
            pl.BlockSpec((1, 1, _TBLK), lambda i: (i, 0, 0)),
            pl.BlockSpec((1, 1, _TBLK), lambda i: (i, 0, 0)),
            pl.BlockSpec((_TBLK, _GW), lambda i: (i, 0)),
            pl.BlockSpec((_TBLK, _GW), lambda i: (i, 0)),
            pl.BlockSpec((_TBLK, _GW), lambda i: (i, 0)),
            full((_D, 150)),
            full((_D, 150)),
            full((_D, 150)),
            full((16, 150)),
            full((1, 150)),
            full((150, 100)),
            full((1, 100)),
            full((1, 100)),
            full((1, 100)),
        ],
        out_specs=pl.BlockSpec((_TBLK, 100), lambda i: (i, 0)),
        out_shape=jax.ShapeDtypeStruct((_N, 100), jnp.float32),
        compiler_params=pltpu.CompilerParams(
            dimension_semantics=("parallel",)),
    )(x2r, x3r, g0, g1, g4, a0, a1, a4, p23, b1, w2, b2, s2, bt2)


def kernel(X, emb0, emb1, emb2, emb3, emb4, gamma0, beta0, W1, bias1,
           gamma1, beta1, W2, bias2, gamma2, beta2):
    inv = jnp.float32(1.0) / jnp.sqrt(jnp.float32(1.0 + 1e-3))
    s0 = gamma0 * inv
    w1e = W1 * s0[:, None]
    b1e = (bias1 + beta0 @ W1).reshape(1, 150)
    a0, a1, a4 = w1e[0:50], w1e[50:100], w1e[107:157]
    p2 = emb2 @ w1e[100:103]
    p3 = emb3 @ w1e[103:107]
    p23 = jnp.concatenate([p2, p3, jnp.zeros((3, 150), jnp.float32)], axis=0)
    w2e = W2 * (gamma1 * inv)[:, None]
    b2e = (bias2 + beta1 @ W2).reshape(1, 100)
    s2 = (gamma2 * inv).reshape(1, 100)
    bt2 = beta2.reshape(1, 100)

    xf = X.reshape(_N, 5)
    x0 = xf[:, 0].reshape(_NW, _CPW, _CHUNK)
    x1 = xf[:, 1].reshape(_NW, _CPW, _CHUNK)
    x4 = xf[:, 4].reshape(_NW, _CPW, _CHUNK)
    g0, g1, g4 = _sc_gather(x0, x1, x4, _prep_table(emb0), _prep_table(emb1),
                            _prep_table(emb4))
    x2r = xf[:, 2].reshape(_NB, 1, _TBLK)
    x3r = xf[:, 3].reshape(_NB, 1, _TBLK)
    out = _mlp(g0, g1, g4, x2r, x3r, a0, a1, a4, p23, b1e, w2e, b2e, s2, bt2)
    return out.reshape(_B, _L, 100)


# speedup vs baseline: 1.6341x; 1.6341x over previous
import functools

import jax
import jax.numpy as jnp
from jax import lax
from jax.experimental import pallas as pl
from jax.experimental.pallas import tpu as pltpu
from jax.experimental.pallas import tpu_sc as plsc

_B, _L = 1024, 200
_N = _B * _L
_D = 50
_DP = 56
_GW = 128
_NC, _NS = 2, 16
_NW = _NC * _NS
_CHUNK = 128
_CPW = _N // (_NW * _CHUNK)
_TBLK = 512
_NB = _N // _TBLK
_PBLK = 131072
_RBLK = 256


def _prep_table(tT, c_blk=1024):
    v = tT.shape[1]
    nb = (v + c_blk - 1) // c_blk

    def body(i_ref, o_ref):
        xt = i_ref[...].T
        o_ref[...] = jnp.concatenate(
            [xt, jnp.zeros((c_blk, _DP - _D), jnp.float32)], axis=1)

    return pl.pallas_call(
        body,
        grid=(nb,),
        in_specs=[pl.BlockSpec((_D, c_blk), lambda i: (0, i))],
        out_specs=pl.BlockSpec((c_blk, _DP), lambda i: (i, 0)),
        out_shape=jax.ShapeDtypeStruct((nb * c_blk, _DP), jnp.float32),
        compiler_params=pltpu.CompilerParams(
            dimension_semantics=("arbitrary",)),
    )(tT)


def _sc_gather(x0, x1, x4, t0, t1, t4):
    mesh = plsc.VectorSubcoreMesh(core_axis_name="c", subcore_axis_name="s")
    ot = [jax.ShapeDtypeStruct((_N, _GW), jnp.float32) for _ in range(3)]

    @functools.partial(
        pl.kernel, mesh=mesh, out_type=ot,
        compiler_params=pltpu.CompilerParams(use_tc_tiling_on_sc=False),
        scratch_types=[
            pltpu.VMEM((_CPW, _CHUNK), jnp.int32),
            pltpu.VMEM((2, _CHUNK, _DP), jnp.float32),
            pltpu.SemaphoreType.DMA,
        ],
    )
    def k(x0h, x1h, x4h, t0h, t1h, t4h, o0h, o1h, o4h,
          iv, bufs, sem):
        wid = lax.axis_index("s") * _NC + lax.axis_index("c")
        row0 = wid * _CPW
        for xh, th, ohbm in ((x0h, t0h, o0h), (x1h, t1h, o1h), (x4h, t4h, o4h)):
            pltpu.sync_copy(xh.at[wid], iv)

            def ga(j, b, th=th):
                return pltpu.make_async_copy(th.at[iv.at[j]], bufs.at[b], sem)

            ga(0, 0).start()

            def body(j, carry, ga=ga, ohbm=ohbm):
                b = lax.rem(j, 2)
                ga(j, b).wait()
                nxt = lax.min(j + 1, _CPW - 1)
                ga(nxt, 1 - b).start()
                pltpu.sync_copy(
                    bufs.at[b],
                    ohbm.at[pl.ds((row0 + j) * _CHUNK, _CHUNK),
                            pl.ds(0, _DP)])
                return carry

            lax.fori_loop(0, _CPW, body, 0)
            ga(_CPW - 1, _CPW % 2).wait()

    return k(x0, x1, x4, t0, t1, t4)


def _mlp(g0, g1, g4, x2r, x3r, a0, a1, a4, p23, b1, w2, b2, s2, bt2):
    def body(x2_ref, x3_ref, g0_ref, g1_ref, g4_ref, a0_ref, a1_ref, a4_ref,
             p23_ref, b1_ref, w2_ref, b2_ref, s2_ref, bt2_ref, o_ref):
        x2 = x2_ref[0, 0, :]
        x3 = x3_ref[0, 0, :]
        it = lax.broadcasted_iota(jnp.int32, (_TBLK, 16), 1)
        oh = jnp.logical_or(x2[:, None] == it,
                            (x3[:, None] + 5) == it).astype(jnp.float32)
        acc = jnp.dot(g0_ref[:, 0:_D], a0_ref[...],
                      preferred_element_type=jnp.float32)
        acc += jnp.dot(g1_ref[:, 0:_D], a1_ref[...],
                       preferred_element_type=jnp.float32)
        acc += jnp.dot(g4_ref[:, 0:_D], a4_ref[...],
                       preferred_element_type=jnp.float32)
        acc += jnp.dot(oh, p23_ref[...], preferred_element_type=jnp.float32)
        acc += b1_ref[...]
        h = jnp.where(acc > 0, acc, jnp.exp(acc) - 1.0)
        acc2 = jnp.dot(h, w2_ref[...], preferred_element_type=jnp.float32)
        acc2 += b2_ref[...]
        h2 = jnp.where(acc2 > 0, acc2, jnp.exp(acc2) - 1.0)
        o_ref[...] = h2 * s2_ref[...] + bt2_ref[...]

    full = lambda shape: pl.BlockSpec(shape, lambda i: tuple(0 for _ in shape))
    return pl.pallas_call(
        body,
        grid=(_NB,),
        in_specs=[
            pl.BlockSpec((1, 1, _TBLK), lambda i: (i, 0, 0)),
            pl.BlockSpec((1, 1, _TBLK), lambda i: (i, 0, 0)),
            pl.BlockSpec((_TBLK, _GW), lambda i: (i, 0)),
            pl.BlockSpec((_TBLK, _GW), lambda i: (i, 0)),
            pl.BlockSpec((_TBLK, _GW), lambda i: (i, 0)),
            full((_D, 150)),
            full((_D, 150)),
            full((_D, 150)),
            full((16, 150)),
            full((1, 150)),
            full((150, 100)),
            full((1, 100)),
            full((1, 100)),
            full((1, 100)),
        ],
        out_specs=pl.BlockSpec((_TBLK, 100), lambda i: (i, 0)),
        out_shape=jax.ShapeDtypeStruct((_N, 100), jnp.float32),
        compiler_params=pltpu.CompilerParams(
            dimension_semantics=("parallel",)),
    )(x2r, x3r, g0, g1, g4, a0, a1, a4, p23, b1, w2, b2, s2, bt2)


def kernel(X, emb0, emb1, emb2, emb3, emb4, gamma0, beta0, W1, bias1,
           gamma1, beta1, W2, bias2, gamma2, beta2):
    inv = jnp.float32(1.0) / jnp.sqrt(jnp.float32(1.0 + 1e-3))
    s0 = gamma0 * inv
    w1e = W1 * s0[:, None]
    b1e = (bias1 + beta0 @ W1).reshape(1, 150)
    a0, a1, a4 = w1e[0:50], w1e[50:100], w1e[107:157]
    p2 = emb2 @ w1e[100:103]
    p3 = emb3 @ w1e[103:107]
    p23 = jnp.concatenate([p2, p3, jnp.zeros((3, 150), jnp.float32)], axis=0)
    w2e = W2 * (gamma1 * inv)[:, None]
    b2e = (bias2 + beta1 @ W2).reshape(1, 100)
    s2 = (gamma2 * inv).reshape(1, 100)
    bt2 = beta2.reshape(1, 100)

    xf = X.reshape(_N, 5)
    x0 = xf[:, 0].reshape(_NW, _CPW, _CHUNK)
    x1 = xf[:, 1].reshape(_NW, _CPW, _CHUNK)
    x4 = xf[:, 4].reshape(_NW, _CPW, _CHUNK)
    g0, g1, g4 = _sc_gather(x0, x1, x4, _prep_table(emb0.T),
                            _prep_table(emb1.T), _prep_table(emb4.T))
    x2r = xf[:, 2].reshape(_NB, 1, _TBLK)
    x3r = xf[:, 3].reshape(_NB, 1, _TBLK)
    out = _mlp(g0, g1, g4, x2r, x3r, a0, a1, a4, p23, b1e, w2e, b2e, s2, bt2)
    return out.reshape(_B, _L, 100)


# speedup vs baseline: 2.4053x; 1.4719x over previous
import functools

import jax
import jax.numpy as jnp
from jax import lax
from jax.experimental import pallas as pl
from jax.experimental.pallas import tpu as pltpu
from jax.experimental.pallas import tpu_sc as plsc

_B, _L = 1024, 200
_N = _B * _L
_D = 50
_DP = 56
_GW = 128
_NC, _NS = 2, 16
_NW = _NC * _NS
_CHUNK = 128
_CPW = _N // (_NW * _CHUNK)
_TBLK = 2048
_NB = _N // _TBLK
_PBLK = 131072
_RBLK = 256


def _prep_table(tT, c_blk=8192):
    v = tT.shape[1]
    nb = (v + c_blk - 1) // c_blk

    def body(i_ref, o_ref):
        xt = i_ref[...].T
        o_ref[...] = jnp.concatenate(
            [xt, jnp.zeros((c_blk, _DP - _D), jnp.float32)], axis=1)

    return pl.pallas_call(
        body,
        grid=(nb,),
        in_specs=[pl.BlockSpec((_D, c_blk), lambda i: (0, i))],
        out_specs=pl.BlockSpec((c_blk, _DP), lambda i: (i, 0)),
        out_shape=jax.ShapeDtypeStruct((nb * c_blk, _DP), jnp.float32),
        compiler_params=pltpu.CompilerParams(
            dimension_semantics=("arbitrary",)),
    )(tT)


def _sc_gather(x0, x1, x4, t0, t1, t4):
    mesh = plsc.VectorSubcoreMesh(core_axis_name="c", subcore_axis_name="s")
    ot = [jax.ShapeDtypeStruct((_N, _GW), jnp.float32) for _ in range(3)]

    @functools.partial(
        pl.kernel, mesh=mesh, out_type=ot,
        compiler_params=pltpu.CompilerParams(use_tc_tiling_on_sc=False),
        scratch_types=[
            pltpu.VMEM((_CPW, _CHUNK), jnp.int32),
            pltpu.VMEM((2, _CHUNK, _DP), jnp.float32),
            pltpu.SemaphoreType.DMA,
        ],
    )
    def k(x0h, x1h, x4h, t0h, t1h, t4h, o0h, o1h, o4h,
          iv, bufs, sem):
        wid = lax.axis_index("s") * _NC + lax.axis_index("c")
        row0 = wid * _CPW
        for xh, th, ohbm in ((x0h, t0h, o0h), (x1h, t1h, o1h), (x4h, t4h, o4h)):
            pltpu.sync_copy(xh.at[wid], iv)

            def ga(j, b, th=th):
                return pltpu.make_async_copy(th.at[iv.at[j]], bufs.at[b], sem)

            ga(0, 0).start()

            def body(j, carry, ga=ga, ohbm=ohbm):
                b = lax.rem(j, 2)
                ga(j, b).wait()
                nxt = lax.min(j + 1, _CPW - 1)
                ga(nxt, 1 - b).start()
                pltpu.sync_copy(
                    bufs.at[b],
                    ohbm.at[pl.ds((row0 + j) * _CHUNK, _CHUNK),
                            pl.ds(0, _DP)])
                return carry

            lax.fori_loop(0, _CPW, body, 0)
            ga(_CPW - 1, _CPW % 2).wait()

    return k(x0, x1, x4, t0, t1, t4)


def _mlp(g0, g1, g4, x2r, x3r, a0, a1, a4, p23, b1, w2, b2, s2, bt2):
    def body(x2_ref, x3_ref, g0_ref, g1_ref, g4_ref, a0_ref, a1_ref, a4_ref,
             p23_ref, b1_ref, w2_ref, b2_ref, s2_ref, bt2_ref, o_ref):
        x2 = x2_ref[0, 0, :]
        x3 = x3_ref[0, 0, :]
        it = lax.broadcasted_iota(jnp.int32, (_TBLK, 16), 1)
        oh = jnp.logical_or(x2[:, None] == it,
                            (x3[:, None] + 5) == it).astype(jnp.float32)
        acc = jnp.dot(g0_ref[:, 0:_D], a0_ref[...],
                      preferred_element_type=jnp.float32)
        acc += jnp.dot(g1_ref[:, 0:_D], a1_ref[...],
                       preferred_element_type=jnp.float32)
        acc += jnp.dot(g4_ref[:, 0:_D], a4_ref[...],
                       preferred_element_type=jnp.float32)
        acc += jnp.dot(oh, p23_ref[...], preferred_element_type=jnp.float32)
        acc += b1_ref[...]
        h = jnp.where(acc > 0, acc, jnp.exp(acc) - 1.0)
        acc2 = jnp.dot(h, w2_ref[...], preferred_element_type=jnp.float32)
        acc2 += b2_ref[...]
        h2 = jnp.where(acc2 > 0, acc2, jnp.exp(acc2) - 1.0)
        o_ref[...] = h2 * s2_ref[...] + bt2_ref[...]

    full = lambda shape: pl.BlockSpec(shape, lambda i: tuple(0 for _ in shape))
    return pl.pallas_call(
        body,
        grid=(_NB,),
        in_specs=[
            pl.BlockSpec((1, 1, _TBLK), lambda i: (i, 0, 0)),
            pl.BlockSpec((1, 1, _TBLK), lambda i: (i, 0, 0)),
            pl.BlockSpec((_TBLK, _GW), lambda i: (i, 0)),
            pl.BlockSpec((_TBLK, _GW), lambda i: (i, 0)),
            pl.BlockSpec((_TBLK, _GW), lambda i: (i, 0)),
            full((_D, 150)),
            full((_D, 150)),
            full((_D, 150)),
            full((16, 150)),
            full((1, 150)),
            full((150, 100)),
            full((1, 100)),
            full((1, 100)),
            full((1, 100)),
        ],
        out_specs=pl.BlockSpec((_TBLK, 100), lambda i: (i, 0)),
        out_shape=jax.ShapeDtypeStruct((_N, 100), jnp.float32),
        compiler_params=pltpu.CompilerParams(
            dimension_semantics=("parallel",)),
    )(x2r, x3r, g0, g1, g4, a0, a1, a4, p23, b1, w2, b2, s2, bt2)


def kernel(X, emb0, emb1, emb2, emb3, emb4, gamma0, beta0, W1, bias1,
           gamma1, beta1, W2, bias2, gamma2, beta2):
    inv = jnp.float32(1.0) / jnp.sqrt(jnp.float32(1.0 + 1e-3))
    s0 = gamma0 * inv
    w1e = W1 * s0[:, None]
    b1e = (bias1 + beta0 @ W1).reshape(1, 150)
    a0, a1, a4 = w1e[0:50], w1e[50:100], w1e[107:157]
    p2 = emb2 @ w1e[100:103]
    p3 = emb3 @ w1e[103:107]
    p23 = jnp.concatenate([p2, p3, jnp.zeros((3, 150), jnp.float32)], axis=0)
    w2e = W2 * (gamma1 * inv)[:, None]
    b2e = (bias2 + beta1 @ W2).reshape(1, 100)
    s2 = (gamma2 * inv).reshape(1, 100)
    bt2 = beta2.reshape(1, 100)

    xf = X.reshape(_N, 5)
    x0 = xf[:, 0].reshape(_NW, _CPW, _CHUNK)
    x1 = xf[:, 1].reshape(_NW, _CPW, _CHUNK)
    x4 = xf[:, 4].reshape(_NW, _CPW, _CHUNK)
    g0, g1, g4 = _sc_gather(x0, x1, x4, _prep_table(emb0.T),
                            _prep_table(emb1.T), _prep_table(emb4.T))
    x2r = xf[:, 2].reshape(_NB, 1, _TBLK)
    x3r = xf[:, 3].reshape(_NB, 1, _TBLK)
    out = _mlp(g0, g1, g4, x2r, x3r, a0, a1, a4, p23, b1e, w2e, b2e, s2, bt2)
    return out.reshape(_B, _L, 100)


# speedup vs baseline: 2.4667x; 1.0255x over previous
import functools

import jax
import jax.numpy as jnp
from jax import lax
from jax.experimental import pallas as pl
from jax.experimental.pallas import tpu as pltpu
from jax.experimental.pallas import tpu_sc as plsc

_B, _L = 1024, 200
_N = _B * _L
_D = 50
_DP = 56
_GW = 128
_NC, _NS = 2, 16
_NW = _NC * _NS
_CHUNK = 128
_CPW = _N // (_NW * _CHUNK)
_TBLK = 4096
_NB = _N // _TBLK
_PBLK = 131072
_RBLK = 256


def _prep_table(tT, c_blk=16384):
    v = tT.shape[1]
    nb = (v + c_blk - 1) // c_blk

    def body(i_ref, o_ref):
        xt = i_ref[...].T
        o_ref[...] = jnp.concatenate(
            [xt, jnp.zeros((c_blk, _DP - _D), jnp.float32)], axis=1)

    return pl.pallas_call(
        body,
        grid=(nb,),
        in_specs=[pl.BlockSpec((_D, c_blk), lambda i: (0, i))],
        out_specs=pl.BlockSpec((c_blk, _DP), lambda i: (i, 0)),
        out_shape=jax.ShapeDtypeStruct((nb * c_blk, _DP), jnp.float32),
        compiler_params=pltpu.CompilerParams(
            dimension_semantics=("arbitrary",)),
    )(tT)


def _sc_gather(x0, x1, x4, t0, t1, t4):
    mesh = plsc.VectorSubcoreMesh(core_axis_name="c", subcore_axis_name="s")
    ot = [jax.ShapeDtypeStruct((_N, _GW), jnp.float32) for _ in range(3)]

    @functools.partial(
        pl.kernel, mesh=mesh, out_type=ot,
        compiler_params=pltpu.CompilerParams(use_tc_tiling_on_sc=False),
        scratch_types=[
            pltpu.VMEM((_CPW, _CHUNK), jnp.int32),
            pltpu.VMEM((2, _CHUNK, _DP), jnp.float32),
            pltpu.SemaphoreType.DMA,
        ],
    )
    def k(x0h, x1h, x4h, t0h, t1h, t4h, o0h, o1h, o4h,
          iv, bufs, sem):
        wid = lax.axis_index("s") * _NC + lax.axis_index("c")
        row0 = wid * _CPW
        for xh, th, ohbm in ((x0h, t0h, o0h), (x1h, t1h, o1h), (x4h, t4h, o4h)):
            pltpu.sync_copy(xh.at[wid], iv)

            def ga(j, b, th=th):
                return pltpu.make_async_copy(th.at[iv.at[j]], bufs.at[b], sem)

            ga(0, 0).start()

            def body(j, carry, ga=ga, ohbm=ohbm):
                b = lax.rem(j, 2)
                ga(j, b).wait()
                nxt = lax.min(j + 1, _CPW - 1)
                ga(nxt, 1 - b).start()
                pltpu.sync_copy(
                    bufs.at[b],
                    ohbm.at[pl.ds((row0 + j) * _CHUNK, _CHUNK),
                            pl.ds(0, _DP)])
                return carry

            lax.fori_loop(0, _CPW, body, 0)
            ga(_CPW - 1, _CPW % 2).wait()

    return k(x0, x1, x4, t0, t1, t4)


def _mlp(g0, g1, g4, x2r, x3r, a0, a1, a4, p23, b1, w2, b2, s2, bt2):
    def body(x2_ref, x3_ref, g0_ref, g1_ref, g4_ref, a0_ref, a1_ref, a4_ref,
             p23_ref, b1_ref, w2_ref, b2_ref, s2_ref, bt2_ref, o_ref):
        x2 = x2_ref[0, 0, :]
        x3 = x3_ref[0, 0, :]
        it = lax.broadcasted_iota(jnp.int32, (_TBLK, 16), 1)
        oh = jnp.logical_or(x2[:, None] == it,
                            (x3[:, None] + 5) == it).astype(jnp.float32)
        acc = jnp.dot(g0_ref[:, 0:_D], a0_ref[...],
                      preferred_element_type=jnp.float32)
        acc += jnp.dot(g1_ref[:, 0:_D], a1_ref[...],
                       preferred_element_type=jnp.float32)
        acc += jnp.dot(g4_ref[:, 0:_D], a4_ref[...],
                       preferred_element_type=jnp.float32)
        acc += jnp.dot(oh, p23_ref[...], preferred_element_type=jnp.float32)
        acc += b1_ref[...]
        h = jnp.where(acc > 0, acc, jnp.exp(acc) - 1.0)
        acc2 = jnp.dot(h, w2_ref[...], preferred_element_type=jnp.float32)
        acc2 += b2_ref[...]
        h2 = jnp.where(acc2 > 0, acc2, jnp.exp(acc2) - 1.0)
        o_ref[...] = h2 * s2_ref[...] + bt2_ref[...]

    full = lambda shape: pl.BlockSpec(shape, lambda i: tuple(0 for _ in shape))
    return pl.pallas_call(
        body,
        grid=(_NB,),
        in_specs=[
            pl.BlockSpec((1, 1, _TBLK), lambda i: (i, 0, 0)),
            pl.BlockSpec((1, 1, _TBLK), lambda i: (i, 0, 0)),
            pl.BlockSpec((_TBLK, _GW), lambda i: (i, 0)),
            pl.BlockSpec((_TBLK, _GW), lambda i: (i, 0)),
            pl.BlockSpec((_TBLK, _GW), lambda i: (i, 0)),
            full((_D, 150)),
            full((_D, 150)),
            full((_D, 150)),
            full((16, 150)),
            full((1, 150)),
            full((150, 100)),
            full((1, 100)),
            full((1, 100)),
            full((1, 100)),
        ],
        out_specs=pl.BlockSpec((_TBLK, 100), lambda i: (i, 0)),
        out_shape=jax.ShapeDtypeStruct((_N, 100), jnp.float32),
        compiler_params=pltpu.CompilerParams(
            dimension_semantics=("parallel",)),
    )(x2r, x3r, g0, g1, g4, a0, a1, a4, p23, b1, w2, b2, s2, bt2)


def kernel(X, emb0, emb1, emb2, emb3, emb4, gamma0, beta0, W1, bias1,
           gamma1, beta1, W2, bias2, gamma2, beta2):
    inv = jnp.float32(1.0) / jnp.sqrt(jnp.float32(1.0 + 1e-3))
    s0 = gamma0 * inv
    w1e = W1 * s0[:, None]
    b1e = (bias1 + beta0 @ W1).reshape(1, 150)
    a0, a1, a4 = w1e[0:50], w1e[50:100], w1e[107:157]
    p2 = emb2 @ w1e[100:103]
    p3 = emb3 @ w1e[103:107]
    p23 = jnp.concatenate([p2, p3, jnp.zeros((3, 150), jnp.float32)], axis=0)
    w2e = W2 * (gamma1 * inv)[:, None]
    b2e = (bias2 + beta1 @ W2).reshape(1, 100)
    s2 = (gamma2 * inv).reshape(1, 100)
    bt2 = beta2.reshape(1, 100)

    xf = X.reshape(_N, 5)
    x0 = xf[:, 0].reshape(_NW, _CPW, _CHUNK)
    x1 = xf[:, 1].reshape(_NW, _CPW, _CHUNK)
    x4 = xf[:, 4].reshape(_NW, _CPW, _CHUNK)
    g0, g1, g4 = _sc_gather(x0, x1, x4, _prep_table(emb0.T),
                            _prep_table(emb1.T), _prep_table(emb4.T))
    x2r = xf[:, 2].reshape(_NB, 1, _TBLK)
    x3r = xf[:, 3].reshape(_NB, 1, _TBLK)
    out = _mlp(g0, g1, g4, x2r, x3r, a0, a1, a4, p23, b1e, w2e, b2e, s2, bt2)
    return out.reshape(_B, _L, 100)


# speedup vs baseline: 2.5484x; 1.0331x over previous
import functools

import jax
import jax.numpy as jnp
from jax import lax
from jax.experimental import pallas as pl
from jax.experimental.pallas import tpu as pltpu
from jax.experimental.pallas import tpu_sc as plsc

_B, _L = 1024, 200
_N = _B * _L
_D = 50
_DP = 56
_GW = 128
_NC, _NS = 2, 16
_NW = _NC * _NS
_CHUNK = 128
_CPW = _N // (_NW * _CHUNK)
_TBLK = 4096
_NB = _N // _TBLK
_PBLK = 131072
_RBLK = 256


def _prep_table(tT, c_blk=16384):
    v = tT.shape[1]
    nb = (v + c_blk - 1) // c_blk

    def body(i_ref, o_ref):
        xt = i_ref[...].T
        o_ref[...] = jnp.concatenate(
            [xt, jnp.zeros((c_blk, _DP - _D), jnp.float32)], axis=1)

    return pl.pallas_call(
        body,
        grid=(nb,),
        in_specs=[pl.BlockSpec((_D, c_blk), lambda i: (0, i))],
        out_specs=pl.BlockSpec((c_blk, _DP), lambda i: (i, 0)),
        out_shape=jax.ShapeDtypeStruct((nb * c_blk, _DP), jnp.float32),
        compiler_params=pltpu.CompilerParams(
            dimension_semantics=("arbitrary",)),
    )(tT)


def _sc_gather1(x, t):
    mesh = plsc.VectorSubcoreMesh(core_axis_name="c", subcore_axis_name="s")

    @functools.partial(
        pl.kernel, mesh=mesh,
        out_type=jax.ShapeDtypeStruct((_N, _GW), jnp.float32),
        compiler_params=pltpu.CompilerParams(use_tc_tiling_on_sc=False),
        scratch_types=[
            pltpu.VMEM((_CPW, _CHUNK), jnp.int32),
            pltpu.VMEM((2, _CHUNK, _DP), jnp.float32),
            pltpu.SemaphoreType.DMA,
        ],
    )
    def k(xh, th, ohbm, iv, bufs, sem):
        wid = lax.axis_index("s") * _NC + lax.axis_index("c")
        row0 = wid * _CPW
        pltpu.sync_copy(xh.at[wid], iv)

        def ga(j, b):
            return pltpu.make_async_copy(th.at[iv.at[j]], bufs.at[b], sem)

        ga(0, 0).start()

        def body(j, carry):
            b = lax.rem(j, 2)
            ga(j, b).wait()
            nxt = lax.min(j + 1, _CPW - 1)
            ga(nxt, 1 - b).start()
            pltpu.sync_copy(
                bufs.at[b],
                ohbm.at[pl.ds((row0 + j) * _CHUNK, _CHUNK), pl.ds(0, _DP)])
            return carry

        lax.fori_loop(0, _CPW, body, 0)
        ga(_CPW - 1, _CPW % 2).wait()

    return k(x, t)


def _mlp(g0, g1, g4, x2r, x3r, a0, a1, a4, p23, b1, w2, b2, s2, bt2):
    def body(x2_ref, x3_ref, g0_ref, g1_ref, g4_ref, a0_ref, a1_ref, a4_ref,
             p23_ref, b1_ref, w2_ref, b2_ref, s2_ref, bt2_ref, o_ref):
        x2 = x2_ref[0, 0, :]
        x3 = x3_ref[0, 0, :]
        it = lax.broadcasted_iota(jnp.int32, (_TBLK, 16), 1)
        oh = jnp.logical_or(x2[:, None] == it,
                            (x3[:, None] + 5) == it).astype(jnp.float32)
        acc = jnp.dot(g0_ref[:, 0:_D], a0_ref[...],
                      preferred_element_type=jnp.float32)
        acc += jnp.dot(g1_ref[:, 0:_D], a1_ref[...],
                       preferred_element_type=jnp.float32)
        acc += jnp.dot(g4_ref[:, 0:_D], a4_ref[...],
                       preferred_element_type=jnp.float32)
        acc += jnp.dot(oh, p23_ref[...], preferred_element_type=jnp.float32)
        acc += b1_ref[...]
        h = jnp.where(acc > 0, acc, jnp.exp(acc) - 1.0)
        acc2 = jnp.dot(h, w2_ref[...], preferred_element_type=jnp.float32)
        acc2 += b2_ref[...]
        h2 = jnp.where(acc2 > 0, acc2, jnp.exp(acc2) - 1.0)
        o_ref[...] = h2 * s2_ref[...] + bt2_ref[...]

    full = lambda shape: pl.BlockSpec(shape, lambda i: tuple(0 for _ in shape))
    return pl.pallas_call(
        body,
        grid=(_NB,),
        in_specs=[
            pl.BlockSpec((1, 1, _TBLK), lambda i: (i, 0, 0)),
            pl.BlockSpec((1, 1, _TBLK), lambda i: (i, 0, 0)),
            pl.BlockSpec((_TBLK, _GW), lambda i: (i, 0)),
            pl.BlockSpec((_TBLK, _GW), lambda i: (i, 0)),
            pl.BlockSpec((_TBLK, _GW), lambda i: (i, 0)),
            full((_D, 150)),
            full((_D, 150)),
            full((_D, 150)),
            full((16, 150)),
            full((1, 150)),
            full((150, 100)),
            full((1, 100)),
            full((1, 100)),
            full((1, 100)),
        ],
        out_specs=pl.BlockSpec((_TBLK, 100), lambda i: (i, 0)),
        out_shape=jax.ShapeDtypeStruct((_N, 100), jnp.float32),
        compiler_params=pltpu.CompilerParams(
            dimension_semantics=("parallel",)),
    )(x2r, x3r, g0, g1, g4, a0, a1, a4, p23, b1, w2, b2, s2, bt2)


def kernel(X, emb0, emb1, emb2, emb3, emb4, gamma0, beta0, W1, bias1,
           gamma1, beta1, W2, bias2, gamma2, beta2):
    inv = jnp.float32(1.0) / jnp.sqrt(jnp.float32(1.0 + 1e-3))
    s0 = gamma0 * inv
    w1e = W1 * s0[:, None]
    b1e = (bias1 + beta0 @ W1).reshape(1, 150)
    a0, a1, a4 = w1e[0:50], w1e[50:100], w1e[107:157]
    p2 = emb2 @ w1e[100:103]
    p3 = emb3 @ w1e[103:107]
    p23 = jnp.concatenate([p2, p3, jnp.zeros((3, 150), jnp.float32)], axis=0)
    w2e = W2 * (gamma1 * inv)[:, None]
    b2e = (bias2 + beta1 @ W2).reshape(1, 100)
    s2 = (gamma2 * inv).reshape(1, 100)
    bt2 = beta2.reshape(1, 100)

    xf = X.reshape(_N, 5)
    x0 = xf[:, 0].reshape(_NW, _CPW, _CHUNK)
    x1 = xf[:, 1].reshape(_NW, _CPW, _CHUNK)
    x4 = xf[:, 4].reshape(_NW, _CPW, _CHUNK)
    g0 = _sc_gather1(x0, _prep_table(emb0.T))
    g1 = _sc_gather1(x1, _prep_table(emb1.T))
    g4 = _sc_gather1(x4, _prep_table(emb4.T))
    x2r = xf[:, 2].reshape(_NB, 1, _TBLK)
    x3r = xf[:, 3].reshape(_NB, 1, _TBLK)
    out = _mlp(g0, g1, g4, x2r, x3r, a0, a1, a4, p23, b1e, w2e, b2e, s2, bt2)
    return out.reshape(_B, _L, 100)


# speedup vs baseline: 2.5489x; 1.0002x over previous
import functools

import jax
import jax.numpy as jnp
from jax import lax
from jax.experimental import pallas as pl
from jax.experimental.pallas import tpu as pltpu
from jax.experimental.pallas import tpu_sc as plsc

_B, _L = 1024, 200
_N = _B * _L
_D = 50
_DP = 56
_GW = 128
_NC, _NS = 2, 16
_NW = _NC * _NS
_CHUNK = 128
_CPW = _N // (_NW * _CHUNK)
_TBLK = 4096
_NB = _N // _TBLK


def _prep_table(tT, c_blk=16384):
    v = tT.shape[1]
    nb = (v + c_blk - 1) // c_blk

    def body(i_ref, o_ref):
        xt = i_ref[...].T
        o_ref[...] = jnp.concatenate(
            [xt, jnp.zeros((c_blk, _DP - _D), jnp.float32)], axis=1)

    return pl.pallas_call(
        body,
        grid=(nb,),
        in_specs=[pl.BlockSpec((_D, c_blk), lambda i: (0, i))],
        out_specs=pl.BlockSpec((c_blk, _DP), lambda i: (i, 0)),
        out_shape=jax.ShapeDtypeStruct((nb * c_blk, _DP), jnp.float32),
        compiler_params=pltpu.CompilerParams(
            dimension_semantics=("arbitrary",)),
    )(tT)


def _sc_gather1(x, t):
    mesh = plsc.VectorSubcoreMesh(core_axis_name="c", subcore_axis_name="s")

    @functools.partial(
        pl.kernel, mesh=mesh,
        out_type=jax.ShapeDtypeStruct((_N, _GW), jnp.float32),
        compiler_params=pltpu.CompilerParams(use_tc_tiling_on_sc=False),
        scratch_types=[
            pltpu.VMEM((_CPW, _CHUNK), jnp.int32),
            pltpu.VMEM((2, _CHUNK, _DP), jnp.float32),
            pltpu.SemaphoreType.DMA,
        ],
    )
    def k(xh, th, ohbm, iv, bufs, sem):
        wid = lax.axis_index("s") * _NC + lax.axis_index("c")
        row0 = wid * _CPW
        pltpu.sync_copy(xh.at[wid], iv)

        def ga(j, b):
            return pltpu.make_async_copy(th.at[iv.at[j]], bufs.at[b], sem)

        ga(0, 0).start()

        def body(j, carry):
            b = lax.rem(j, 2)
            ga(j, b).wait()
            nxt = lax.min(j + 1, _CPW - 1)
            ga(nxt, 1 - b).start()
            pltpu.sync_copy(
                bufs.at[b],
                ohbm.at[pl.ds((row0 + j) * _CHUNK, _CHUNK), pl.ds(0, _DP)])
            return carry

        lax.fori_loop(0, _CPW, body, 0)
        ga(_CPW - 1, _CPW % 2).wait()

    return k(x, t)


def _mlp(g0, g1, g4, x2r, x3r, a0, a1, a4, p23, b1, w2, b2, s2, bt2):
    def body(x2_ref, x3_ref, g0_ref, g1_ref, g4_ref, a0_ref, a1_ref, a4_ref,
             p23_ref, b1_ref, w2_ref, b2_ref, s2_ref, bt2_ref, o_ref):
        x2 = x2_ref[0, 0, :]
        x3 = x3_ref[0, 0, :]
        it = lax.broadcasted_iota(jnp.int32, (_TBLK, 16), 1)
        oh = jnp.logical_or(x2[:, None] == it,
                            (x3[:, None] + 5) == it).astype(jnp.float32)
        acc = jnp.dot(g0_ref[:, 0:_D], a0_ref[...],
                      preferred_element_type=jnp.float32)
        acc += jnp.dot(g1_ref[:, 0:_D], a1_ref[...],
                       preferred_element_type=jnp.float32)
        acc += jnp.dot(g4_ref[:, 0:_D], a4_ref[...],
                       preferred_element_type=jnp.float32)
        acc += jnp.dot(oh, p23_ref[...], preferred_element_type=jnp.float32)
        acc += b1_ref[...]
        h = jnp.where(acc > 0, acc, jnp.exp(acc) - 1.0)
        acc2 = jnp.dot(h, w2_ref[...], preferred_element_type=jnp.float32)
        acc2 += b2_ref[...]
        h2 = jnp.where(acc2 > 0, acc2, jnp.exp(acc2) - 1.0)
        o_ref[...] = h2 * s2_ref[...] + bt2_ref[...]

    full = lambda shape: pl.BlockSpec(shape, lambda i: tuple(0 for _ in shape))
    return pl.pallas_call(
        body,
        grid=(_NB,),
        in_specs=[
            pl.BlockSpec((1, 1, _TBLK), lambda i: (i, 0, 0)),
            pl.BlockSpec((1, 1, _TBLK), lambda i: (i, 0, 0)),
            pl.BlockSpec((_TBLK, _GW), lambda i: (i, 0)),
            pl.BlockSpec((_TBLK, _GW), lambda i: (i, 0)),
            pl.BlockSpec((_TBLK, _GW), lambda i: (i, 0)),
            full((_D, 150)),
            full((_D, 150)),
            full((_D, 150)),
            full((16, 150)),
            full((1, 150)),
            full((150, 100)),
            full((1, 100)),
            full((1, 100)),
            full((1, 100)),
        ],
        out_specs=pl.BlockSpec((_TBLK, 100), lambda i: (i, 0)),
        out_shape=jax.ShapeDtypeStruct((_N, 100), jnp.float32),
        compiler_params=pltpu.CompilerParams(
            dimension_semantics=("parallel",)),
    )(x2r, x3r, g0, g1, g4, a0, a1, a4, p23, b1, w2, b2, s2, bt2)


def kernel(X, emb0, emb1, emb2, emb3, emb4, gamma0, beta0, W1, bias1,
           gamma1, beta1, W2, bias2, gamma2, beta2):
    inv = jnp.float32(1.0) / jnp.sqrt(jnp.float32(1.0 + 1e-3))
    s0 = gamma0 * inv
    w1e = W1 * s0[:, None]
    b1e = (bias1 + beta0 @ W1).reshape(1, 150)
    a0, a1, a4 = w1e[0:50], w1e[50:100], w1e[107:157]
    p2 = emb2 @ w1e[100:103]
    p3 = emb3 @ w1e[103:107]
    p23 = jnp.concatenate([p2, p3, jnp.zeros((3, 150), jnp.float32)], axis=0)
    w2e = W2 * (gamma1 * inv)[:, None]
    b2e = (bias2 + beta1 @ W2).reshape(1, 100)
    s2 = (gamma2 * inv).reshape(1, 100)
    bt2 = beta2.reshape(1, 100)

    xf = X.reshape(_N, 5)
    x0 = xf[:, 0].reshape(_NW, _CPW, _CHUNK)
    x1 = xf[:, 1].reshape(_NW, _CPW, _CHUNK)
    x4 = xf[:, 4].reshape(_NW, _CPW, _CHUNK)
    g0 = _sc_gather1(x0, _prep_table(emb0.T))
    g1 = _sc_gather1(x1, _prep_table(emb1.T))
    g4 = _sc_gather1(x4, _prep_table(emb4.T))
    x2r = xf[:, 2].reshape(_NB, 1, _TBLK)
    x3r = xf[:, 3].reshape(_NB, 1, _TBLK)
    out = _mlp(g0, g1, g4, x2r, x3r, a0, a1, a4, p23, b1e, w2e, b2e, s2, bt2)
    return out.reshape(_B, _L, 100)
